# baseline (speedup 1.0000x reference)
import functools

import jax
import jax.numpy as jnp
from jax import lax
from jax.experimental import pallas as pl
from jax.experimental.pallas import tpu as pltpu
from jax.experimental.pallas import tpu_sc as plsc

_VOCAB = 1000000
_EMBED = 100
_EPAD = 128
_B = 4096
_MAXLEN = 128
_N = _B * _MAXLEN

_info = plsc.get_sparse_core_info()
_NC, _NS = _info.num_cores, _info.num_subcores
_NW = _NC * _NS
_PER_W = _N // _NW
_CH = 128
_NCH = _PER_W // _CH

_mesh = plsc.VectorSubcoreMesh(core_axis_name="c", subcore_axis_name="s")


@functools.partial(
    pl.kernel,
    mesh=_mesh,
    out_type=jax.ShapeDtypeStruct((_N, _EPAD), jnp.float32),
    scratch_types=[
        pltpu.VMEM((_NCH, _CH), jnp.int32),
        pltpu.VMEM((2, _CH, _EPAD), jnp.float32),
        pltpu.SemaphoreType.DMA,
        pltpu.SemaphoreType.DMA,
    ],
)
def _gather_kernel(table_hbm, idx_hbm, out_hbm, idx_v, rows_v, gsem, osem):
    wid = lax.axis_index("s") * _NC + lax.axis_index("c")
    base = wid * _PER_W
    pltpu.sync_copy(idx_hbm.at[wid], idx_v)
    zero = jnp.int32(0)

    def chunk(c, _):
        c = c.astype(jnp.int32)
        pltpu.async_copy(table_hbm.at[idx_v.at[c]], rows_v.at[zero], gsem).wait()
        pltpu.sync_copy(rows_v.at[zero], out_hbm.at[pl.ds(base + c * _CH, _CH)])
        return _

    lax.fori_loop(jnp.int32(0), jnp.int32(_NCH), chunk, None)


def kernel(table, rev, lab):
    table_p = jnp.pad(table, ((0, 0), (0, _EPAD - _EMBED)))
    idx = rev.astype(jnp.int32).reshape(_NW, _NCH, _CH)
    out = _gather_kernel(table_p, idx)
    out = out[:, :_EMBED].reshape(_B, _MAXLEN * _EMBED)
    return (out, lab)


# speedup vs baseline: 1.8776x; 1.8776x over previous
import functools

import jax
import jax.numpy as jnp
from jax import lax
from jax.experimental import pallas as pl
from jax.experimental.pallas import tpu as pltpu
from jax.experimental.pallas import tpu_sc as plsc

_VOCAB = 1000000
_EMBED = 100
_EPAD = 128
_B = 4096
_MAXLEN = 128
_N = _B * _MAXLEN

_info = plsc.get_sparse_core_info()
_NC, _NS = _info.num_cores, _info.num_subcores
_NW = _NC * _NS
_PER_W = _N // _NW
_CH = 128
_NCH = _PER_W // _CH

_PAD_ROWS = 4000

_mesh = plsc.VectorSubcoreMesh(core_axis_name="c", subcore_axis_name="s")


def _pad_body(in_ref, out_ref):
    out_ref[:, :_EMBED] = in_ref[...]
    out_ref[:, _EMBED:] = jnp.zeros((_PAD_ROWS, _EPAD - _EMBED), jnp.float32)


_pad_tc = pl.pallas_call(
    _pad_body,
    grid=(_VOCAB // _PAD_ROWS,),
    in_specs=[pl.BlockSpec((_PAD_ROWS, _EMBED), lambda i: (i, jnp.int32(0)))],
    out_specs=pl.BlockSpec((_PAD_ROWS, _EPAD), lambda i: (i, jnp.int32(0))),
    out_shape=jax.ShapeDtypeStruct((_VOCAB, _EPAD), jnp.float32),
)


@functools.partial(
    pl.kernel,
    mesh=_mesh,
    out_type=jax.ShapeDtypeStruct((_N, _EPAD), jnp.float32),
    scratch_types=[
        pltpu.VMEM((_NCH, _CH), jnp.int32),
        pltpu.VMEM((2, _CH, _EPAD), jnp.float32),
        pltpu.SemaphoreType.DMA,
        pltpu.SemaphoreType.DMA,
    ],
)
def _gather_kernel(table_hbm, idx_hbm, out_hbm, idx_v, rows_v, gsem, osem):
    wid = lax.axis_index("s") * _NC + lax.axis_index("c")
    base = wid * _PER_W
    pltpu.sync_copy(idx_hbm.at[wid], idx_v)
    zero = jnp.int32(0)

    def chunk(c, _):
        c = c.astype(jnp.int32)
        pltpu.async_copy(table_hbm.at[idx_v.at[c]], rows_v.at[zero], gsem).wait()
        pltpu.sync_copy(rows_v.at[zero], out_hbm.at[pl.ds(base + c * _CH, _CH)])
        return _

    lax.fori_loop(jnp.int32(0), jnp.int32(_NCH), chunk, None)


def kernel(table, rev, lab):
    table_p = _pad_tc(table)
    idx = rev.astype(jnp.int32).reshape(_NW, _NCH, _CH)
    out = _gather_kernel(table_p, idx)
    out = out[:, :_EMBED].reshape(_B, _MAXLEN * _EMBED)
    return (out, lab)
